# baseline (speedup 1.0000x reference)
import jax
import jax.numpy as jnp
import numpy as np
from jax.experimental import pallas as pl
from jax.experimental.pallas import tpu as pltpu

_INV_PI = 0.31830987334251404
_PI_HI = 3.140625
_PI_MID = 0.0009676536
_C1 = 0.99999749
_C3 = -0.16665168
_C5 = 0.0083095146
_C7 = -0.00018447153


def _fast_sin(x):
    kf = jnp.round(x * _INV_PI)
    r = (x - kf * _PI_HI) - kf * _PI_MID
    s = r * r
    p = r * (_C1 + s * (_C3 + s * (_C5 + s * _C7)))
    ki = kf.astype(jnp.int32)
    signbit = jax.lax.shift_left(ki, 31)
    return jax.lax.bitcast_convert_type(
        jax.lax.bitcast_convert_type(p, jnp.int32) ^ signbit, jnp.float32)


def _siren_block(x_ref, y_ref, w_ref, b_ref, out_ref):
    CG = out_ref.shape[1]
    j = pl.program_id(1)
    x = x_ref[0].astype(jnp.bfloat16).astype(jnp.float32)
    y = y_ref[0].astype(jnp.bfloat16).astype(jnp.float32)
    for c in range(CG):
        col = j * CG + c
        pre = x * w_ref[0, col] + y * w_ref[1, col] + b_ref[0, col]
        out_ref[0, c] = _fast_sin(pre)


def kernel(coords, W, b):
    B, N, D = coords.shape
    C = W.shape[1]
    NT = N // 128
    xp = coords[:, :, 0].reshape(B, NT, 128)
    yp = coords[:, :, 1].reshape(B, NT, 128)
    w16 = W.astype(jnp.bfloat16).astype(jnp.float32)
    b2 = b[None, :]

    CG = 8
    grid = (B, C // CG)
    out_t = pl.pallas_call(
        _siren_block,
        grid=grid,
        in_specs=[
            pl.BlockSpec((1, NT, 128), lambda i, j: (i, 0, 0)),
            pl.BlockSpec((1, NT, 128), lambda i, j: (i, 0, 0)),
            pl.BlockSpec(memory_space=pltpu.SMEM),
            pl.BlockSpec(memory_space=pltpu.SMEM),
        ],
        out_specs=pl.BlockSpec((1, CG, NT, 128), lambda i, j: (i, j, 0, 0)),
        out_shape=jax.ShapeDtypeStruct((B, C, NT, 128), jnp.float32),
        compiler_params=pltpu.CompilerParams(
            dimension_semantics=("parallel", "parallel"),
            vmem_limit_bytes=100 * 1024 * 1024),
    )(xp, yp, w16, b2)
    return out_t.reshape(B, C, N).transpose(0, 2, 1)
